# baseline (speedup 1.0000x reference)
import functools

import jax
import jax.numpy as jnp
from jax.experimental import pallas as pl
from jax.experimental.pallas import tpu as pltpu

DIM = 32
NUM_HEADS = 16

NSEM = 16


def _copy_body(ids_ref, maps_ref, out_ref, sems):
    size = out_ref.shape[1]
    total = ids_ref.shape[0]

    def start(i):
        pltpu.make_async_copy(
            maps_ref.at[ids_ref[i]],
            out_ref.at[i // size, i % size],
            sems.at[i % NSEM],
        ).start()

    def wait(i):
        pltpu.make_async_copy(
            maps_ref.at[ids_ref[i]],
            out_ref.at[i // size, i % size],
            sems.at[i % NSEM],
        ).wait()

    def prime(i, c):
        start(i)
        return c

    jax.lax.fori_loop(0, NSEM, prime, 0)

    def step(i, c):
        wait(i - NSEM)
        start(i)
        return c

    jax.lax.fori_loop(NSEM, total, step, 0)

    def drain(i, c):
        wait(total - NSEM + i)
        return c

    jax.lax.fori_loop(0, NSEM, drain, 0)


@functools.partial(jax.jit, static_argnums=(2, 3))
def _gather(ids, maps, batch, size):
    return pl.pallas_call(
        _copy_body,
        in_specs=[
            pl.BlockSpec(memory_space=pltpu.SMEM),
            pl.BlockSpec(memory_space=pltpu.MemorySpace.HBM),
        ],
        out_specs=pl.BlockSpec(memory_space=pltpu.MemorySpace.HBM),
        out_shape=jax.ShapeDtypeStruct(
            (batch, size, NUM_HEADS, DIM, DIM), jnp.float32),
        scratch_shapes=[pltpu.SemaphoreType.DMA((NSEM,))],
    )(ids, maps)


def kernel(position_ids, maps):
    batch, size = position_ids.shape
    ids = position_ids.reshape(-1).astype(jnp.int32)
    return _gather(ids, maps, batch, size)


# speedup vs baseline: 15.2559x; 15.2559x over previous
import functools

import jax
import jax.numpy as jnp
from jax import lax
from jax.experimental import pallas as pl
from jax.experimental.pallas import tpu as pltpu

DIM = 32
NUM_HEADS = 16

K = 8
D = 4


def _copy_body(ids_ref, maps_ref, out_ref, buf, gsem, ssem):
    size = out_ref.shape[1]
    total = ids_ref.shape[0]

    def g_start(i):
        pltpu.make_async_copy(
            maps_ref.at[ids_ref[i]], buf.at[i % K], gsem.at[i % K]).start()

    def g_wait(i):
        pltpu.make_async_copy(
            maps_ref.at[ids_ref[i]], buf.at[i % K], gsem.at[i % K]).wait()

    def s_start(i):
        pltpu.make_async_copy(
            buf.at[i % K], out_ref.at[i // size, i % size],
            ssem.at[i % K]).start()

    def s_wait(i):
        pltpu.make_async_copy(
            buf.at[i % K], out_ref.at[i // size, i % size],
            ssem.at[i % K]).wait()

    def step(i, c):
        @pl.when(i >= K)
        def _():
            s_wait(i - K)

        g_start(i)

        @pl.when(i >= D)
        def _():
            g_wait(i - D)
            s_start(i - D)

        return c

    lax.fori_loop(0, total, step, 0)

    def tail(j, c):
        i = total - D + j
        g_wait(i)
        s_start(i)
        return c

    lax.fori_loop(0, D, tail, 0)

    def drain(j, c):
        s_wait(total - K + j)
        return c

    lax.fori_loop(0, K, drain, 0)


@functools.partial(jax.jit, static_argnums=(2, 3))
def _gather(ids, maps, batch, size):
    return pl.pallas_call(
        _copy_body,
        in_specs=[
            pl.BlockSpec(memory_space=pltpu.SMEM),
            pl.BlockSpec(memory_space=pltpu.MemorySpace.HBM),
        ],
        out_specs=pl.BlockSpec(memory_space=pltpu.MemorySpace.HBM),
        out_shape=jax.ShapeDtypeStruct(
            (batch, size, NUM_HEADS, DIM, DIM), jnp.float32),
        scratch_shapes=[
            pltpu.VMEM((K, NUM_HEADS, DIM, DIM), jnp.float32),
            pltpu.SemaphoreType.DMA((K,)),
            pltpu.SemaphoreType.DMA((K,)),
        ],
    )(ids, maps)


def kernel(position_ids, maps):
    batch, size = position_ids.shape
    ids = position_ids.reshape(-1).astype(jnp.int32)
    return _gather(ids, maps, batch, size)


# speedup vs baseline: 38.5682x; 2.5281x over previous
import functools

import jax
import jax.numpy as jnp
from jax import lax
from jax.experimental import pallas as pl
from jax.experimental.pallas import tpu as pltpu
from jax.experimental.pallas import tpu_sc as plsc

DIM = 32
NUM_HEADS = 16
ROW = NUM_HEADS * DIM * DIM

NC = 2
NS = 16
NW = NC * NS

G = 4
L = 16


def _gather_body(ids_hbm, tab_hbm, out_hbm, idx_v, rows_v, outs_v, gsem, ssem):
    wid = lax.axis_index("s") * NC + lax.axis_index("c")
    total = ids_hbm.shape[0]
    size = out_hbm.shape[2]
    rows_per_w = tab_hbm.shape[0] // NW
    nstages = rows_per_w // G
    d_base = wid * rows_per_w

    pltpu.sync_copy(ids_hbm, idx_v)

    def in_start(i):
        pltpu.async_copy(
            tab_hbm.at[pl.ds(d_base + i * G, G)], rows_v.at[i % 2],
            gsem.at[i % 2])

    def in_wait(i):
        pltpu.make_async_copy(
            tab_hbm.at[pl.ds(d_base + i * G, G)], rows_v.at[i % 2],
            gsem.at[i % 2]).wait()

    def out_start(i):
        for j in range(G):
            for b in range(2):
                pltpu.async_copy(
                    outs_v.at[i % 2, j, pl.ds(b * size, size)],
                    out_hbm.at[b, d_base + i * G + j],
                    ssem.at[i % 2])

    def out_wait(i):
        for j in range(G):
            for b in range(2):
                pltpu.make_async_copy(
                    outs_v.at[i % 2, j, pl.ds(b * size, size)],
                    out_hbm.at[b, d_base + i * G + j],
                    ssem.at[i % 2]).wait()

    in_start(0)

    def stage(i, carry):
        in_wait(i)

        @pl.when(i + 1 < nstages)
        def _():
            in_start(i + 1)

        @pl.when(i >= 2)
        def _():
            out_wait(i - 2)

        rb = rows_v.at[i % 2]
        ob = outs_v.at[i % 2]

        def chunk(k, c):
            idxk = idx_v[pl.ds(k * L, L)]
            for j in range(G):
                rsel = jnp.full((L,), j, jnp.int32)
                v = plsc.load_gather(rb, [rsel, idxk])
                ob[j, pl.ds(k * L, L)] = v
            return c

        lax.fori_loop(0, total // L, chunk, 0, unroll=8)
        out_start(i)
        return carry

    lax.fori_loop(0, nstages, stage, 0)
    out_wait(nstages - 2)
    out_wait(nstages - 1)


@functools.partial(jax.jit, static_argnums=(2, 3))
def _sc_gather(ids, tab_t, batch, size):
    mesh = plsc.VectorSubcoreMesh(core_axis_name="c", subcore_axis_name="s")
    return pl.kernel(
        _gather_body,
        out_type=jax.ShapeDtypeStruct((batch, ROW, size), jnp.float32),
        mesh=mesh,
        compiler_params=pltpu.CompilerParams(needs_layout_passes=False),
        scratch_types=[
            pltpu.VMEM((batch * size,), jnp.int32),
            pltpu.VMEM((2, G, tab_t.shape[1]), jnp.float32),
            pltpu.VMEM((2, G, batch * size), jnp.float32),
            pltpu.SemaphoreType.DMA((2,)),
            pltpu.SemaphoreType.DMA((2,)),
        ],
    )(ids, tab_t)


def kernel(position_ids, maps):
    batch, size = position_ids.shape
    nmaps = maps.shape[0]
    tab_t = maps.reshape(nmaps, ROW).T
    ids = position_ids.reshape(-1).astype(jnp.int32)
    out_t = _sc_gather(ids, tab_t, batch, size)
    out = out_t.reshape(batch, NUM_HEADS, DIM, DIM, size)
    return out.transpose(0, 4, 1, 2, 3)


# speedup vs baseline: 64.1386x; 1.6630x over previous
import functools

import jax
import jax.numpy as jnp
from jax import lax
from jax.experimental import pallas as pl
from jax.experimental.pallas import tpu as pltpu
from jax.experimental.pallas import tpu_sc as plsc

DIM = 32
NUM_HEADS = 16
ROW = NUM_HEADS * DIM * DIM

NC = 2
NS = 16
NW = NC * NS

CH = 1
NB = 4


def _gather_body(idx_hbm, table_hbm, out_hbm, idx_v, bufs, gsem, ssem):
    wid = lax.axis_index("s") * NC + lax.axis_index("c")
    nch = idx_v.shape[0]
    rows_per_w = nch * CH
    workers_per_batch = out_hbm.shape[1] // rows_per_w
    b = wid // workers_per_batch
    s_base = (wid % workers_per_batch) * rows_per_w
    pltpu.sync_copy(idx_hbm.at[wid], idx_v)

    def g_start(c, j):
        pltpu.async_copy(table_hbm.at[idx_v.at[c]], bufs.at[j], gsem.at[j])

    def g_wait(c, j):
        pltpu.make_async_copy(
            table_hbm.at[idx_v.at[c]], bufs.at[j], gsem.at[j]).wait()

    def s_start(c, j):
        pltpu.async_copy(
            bufs.at[j], out_hbm.at[b, pl.ds(s_base + c * CH, CH)], ssem.at[j])

    def s_wait(c, j):
        pltpu.make_async_copy(
            bufs.at[j], out_hbm.at[b, pl.ds(s_base + c * CH, CH)],
            ssem.at[j]).wait()

    for j in range(NB):
        g_start(j, j)
    T = nch // NB

    def body(t, carry):
        c0 = NB * t
        for j in range(NB):
            g_wait(c0 + j, j)
            s_start(c0 + j, j)

        @pl.when(t < T - 1)
        def _():
            for j in range(NB):
                s_wait(c0 + j, j)
                g_start(c0 + NB + j, j)

        return carry

    lax.fori_loop(0, T, body, 0)
    for j in range(NB):
        s_wait(nch - NB + j, j)


@functools.partial(jax.jit, static_argnums=(2, 3))
def _sc_gather(idx3, table2, batch, size):
    total_rows = batch * size
    nchunks = total_rows // (NW * CH)
    mesh = plsc.VectorSubcoreMesh(core_axis_name="c", subcore_axis_name="s")
    return pl.kernel(
        _gather_body,
        out_type=jax.ShapeDtypeStruct((batch, size, ROW), jnp.float32),
        mesh=mesh,
        scratch_types=[
            pltpu.VMEM((nchunks, CH), jnp.int32),
            pltpu.VMEM((NB, CH, ROW), jnp.float32),
            pltpu.SemaphoreType.DMA((NB,)),
            pltpu.SemaphoreType.DMA((NB,)),
        ],
    )(idx3, table2)


def kernel(position_ids, maps):
    batch, size = position_ids.shape
    total = batch * size
    table2 = maps.reshape(maps.shape[0], ROW)
    idx3 = position_ids.reshape(NW, total // (NW * CH), CH).astype(jnp.int32)
    out = _sc_gather(idx3, table2, batch, size)
    out = out.transpose(0, 2, 1)
    out = out.reshape(batch, NUM_HEADS, DIM, DIM, size)
    return out.transpose(0, 4, 1, 2, 3)
